# baseline (speedup 1.0000x reference)
import jax
import jax.numpy as jnp
from jax.experimental import pallas as pl
from jax.experimental.pallas import tpu as pltpu

_N = 1000000
_B = 8
_F = 64
_TILE = 2048
_NT = (_N + _TILE - 1) // _TILE
_NT3 = (_N + 1 + _TILE - 1) // _TILE

_TINY = 1.1754943508222875e-38

_KS0 = 0
_KS1 = 42
_KS2 = (0x1BD11BDA ^ 42) & 0xFFFFFFFF
_ROT0 = (13, 15, 26, 6)
_ROT1 = (17, 29, 16, 24)


def _rotl(x, d):
    return jax.lax.shift_left(x, jnp.uint32(d)) | jax.lax.shift_right_logical(
        x, jnp.uint32(32 - d))


def _threefry_rounds(v0, v1, rots):
    for r in rots:
        v0 = v0 + v1
        v1 = _rotl(v1, r)
        v1 = v0 ^ v1
    return v0, v1


def _threefry_bits(x1):
    v0 = jnp.zeros_like(x1) + jnp.uint32(_KS0)
    v1 = x1 + jnp.uint32(_KS1)
    v0, v1 = _threefry_rounds(v0, v1, _ROT0)
    v0 = v0 + jnp.uint32(_KS1)
    v1 = v1 + jnp.uint32((_KS2 + 1) & 0xFFFFFFFF)
    v0, v1 = _threefry_rounds(v0, v1, _ROT1)
    v0 = v0 + jnp.uint32(_KS2)
    v1 = v1 + jnp.uint32((_KS0 + 2) & 0xFFFFFFFF)
    v0, v1 = _threefry_rounds(v0, v1, _ROT0)
    v0 = v0 + jnp.uint32(_KS0)
    v1 = v1 + jnp.uint32((_KS1 + 3) & 0xFFFFFFFF)
    v0, v1 = _threefry_rounds(v0, v1, _ROT1)
    v0 = v0 + jnp.uint32(_KS1)
    v1 = v1 + jnp.uint32((_KS2 + 4) & 0xFFFFFFFF)
    v0, v1 = _threefry_rounds(v0, v1, _ROT0)
    v0 = v0 + jnp.uint32(_KS2)
    v1 = v1 + jnp.uint32((_KS0 + 5) & 0xFFFFFFFF)
    return v0 ^ v1


def _gumbel_tile(cols):
    rows = jax.lax.broadcasted_iota(jnp.int32, cols.shape, 0)
    flat = rows * _N + cols
    bits = _threefry_bits(flat.astype(jnp.uint32))
    fb = jax.lax.shift_right_logical(bits, jnp.uint32(9)) | jnp.uint32(0x3F800000)
    floats = jax.lax.bitcast_convert_type(fb, jnp.float32) - jnp.float32(1.0)
    tiny = jnp.float32(_TINY)
    u = jnp.maximum(tiny, floats * (jnp.float32(1.0) - tiny) + tiny)
    return -jnp.log(-jnp.log(u))


def _phase1(z_ref, w_ref, b_ref, logits_ref, m_ref, s_ref, ymax_ref, choice_ref):
    i = pl.program_id(0)

    @pl.when(i == 0)
    def _init():
        m_ref[...] = jnp.full((_B, 128), -jnp.inf, jnp.float32)
        s_ref[...] = jnp.zeros((_B, 128), jnp.float32)
        ymax_ref[...] = jnp.full((_B, 128), -jnp.inf, jnp.float32)
        choice_ref[...] = jnp.zeros((_B, 128), jnp.int32)

    logits = jax.lax.dot_general(
        z_ref[...], w_ref[...], (((1,), (1,)), ((), ())),
        preferred_element_type=jnp.float32,
        precision=jax.lax.Precision.DEFAULT)
    logits = logits + b_ref[...][None, :]
    cols = jax.lax.broadcasted_iota(jnp.int32, (_B, _TILE), 1) + i * _TILE
    logits = jnp.where(cols < _N, logits, -jnp.inf)
    logits_ref[...] = logits

    m_old = m_ref[...]
    tile_max = jnp.max(logits, axis=1, keepdims=True)
    m_new = jnp.maximum(m_old, tile_max)
    s_ref[...] = (s_ref[...] * jnp.exp(m_old - m_new)
                  + jnp.sum(jnp.exp(logits - m_new[:, 0:1]), axis=1, keepdims=True))
    m_ref[...] = m_new

    y = logits + _gumbel_tile(cols)
    ty_max = jnp.max(y, axis=1, keepdims=True)
    t_arg = jnp.min(jnp.where(y == ty_max, cols, jnp.int32(2147483647)),
                    axis=1, keepdims=True)
    upd = ty_max > ymax_ref[...][:, 0:1]
    ymax_ref[...] = jnp.where(upd, ty_max, ymax_ref[...])
    choice_ref[...] = jnp.where(upd, t_arg, choice_ref[...])


def _phase2(logits_ref, m_ref, s_ref, logp_ref):
    lse = m_ref[...][:, 0:1] + jnp.log(s_ref[...][:, 0:1])
    logp_ref[...] = logits_ref[...] - lse


def _phase3(choice_ref, prop_ref):
    i = pl.program_id(0)
    cols = jax.lax.broadcasted_iota(jnp.int32, (_B, _TILE), 1) + i * _TILE
    ch = choice_ref[...][:, 0:1]
    prop_ref[...] = jnp.where(
        cols == 0, jnp.float32(0.5),
        jnp.where(cols == ch + 1, jnp.float32(1.0), jnp.float32(0.0)))


def kernel(z, W, b):
    logits, m, s, _ymax, choice = pl.pallas_call(
        _phase1,
        grid=(_NT,),
        in_specs=[
            pl.BlockSpec((_B, _F), lambda i: (0, 0)),
            pl.BlockSpec((_TILE, _F), lambda i: (i, 0)),
            pl.BlockSpec((_TILE,), lambda i: (i,)),
        ],
        out_specs=[
            pl.BlockSpec((_B, _TILE), lambda i: (0, i)),
            pl.BlockSpec((_B, 128), lambda i: (0, 0)),
            pl.BlockSpec((_B, 128), lambda i: (0, 0)),
            pl.BlockSpec((_B, 128), lambda i: (0, 0)),
            pl.BlockSpec((_B, 128), lambda i: (0, 0)),
        ],
        out_shape=[
            jax.ShapeDtypeStruct((_B, _N), jnp.float32),
            jax.ShapeDtypeStruct((_B, 128), jnp.float32),
            jax.ShapeDtypeStruct((_B, 128), jnp.float32),
            jax.ShapeDtypeStruct((_B, 128), jnp.float32),
            jax.ShapeDtypeStruct((_B, 128), jnp.int32),
        ],
    )(z, W, b)

    log_p = pl.pallas_call(
        _phase2,
        grid=(_NT,),
        in_specs=[
            pl.BlockSpec((_B, _TILE), lambda i: (0, i)),
            pl.BlockSpec((_B, 128), lambda i: (0, 0)),
            pl.BlockSpec((_B, 128), lambda i: (0, 0)),
        ],
        out_specs=pl.BlockSpec((_B, _TILE), lambda i: (0, i)),
        out_shape=jax.ShapeDtypeStruct((_B, _N), jnp.float32),
        input_output_aliases={0: 0},
    )(logits, m, s)

    proposal = pl.pallas_call(
        _phase3,
        grid=(_NT3,),
        in_specs=[pl.BlockSpec((_B, 128), lambda i: (0, 0))],
        out_specs=pl.BlockSpec((_B, _TILE), lambda i: (0, i)),
        out_shape=jax.ShapeDtypeStruct((_B, _N + 1), jnp.float32),
    )(choice)

    return proposal, log_p


# speedup vs baseline: 2.0985x; 2.0985x over previous
import jax
import jax.numpy as jnp
from jax.experimental import pallas as pl
from jax.experimental.pallas import tpu as pltpu

_N = 1000000
_B = 8
_F = 64
_T1 = 8192
_NT1 = (_N + _T1 - 1) // _T1
_T2 = 65536
_NT2 = (_N + 1 + _T2 - 1) // _T2
_SW = _T2 * _NT2

_TINY = 1.1754943508222875e-38

_KS0 = 0
_KS1 = 42
_KS2 = (0x1BD11BDA ^ 42) & 0xFFFFFFFF
_ROT0 = (13, 15, 26, 6)
_ROT1 = (17, 29, 16, 24)


def _rotl(x, d):
    return jax.lax.shift_left(x, jnp.uint32(d)) | jax.lax.shift_right_logical(
        x, jnp.uint32(32 - d))


def _threefry_rounds(v0, v1, rots):
    for r in rots:
        v0 = v0 + v1
        v1 = _rotl(v1, r)
        v1 = v0 ^ v1
    return v0, v1


def _threefry_bits(x1):
    v0 = jnp.zeros_like(x1) + jnp.uint32(_KS0)
    v1 = x1 + jnp.uint32(_KS1)
    v0, v1 = _threefry_rounds(v0, v1, _ROT0)
    v0 = v0 + jnp.uint32(_KS1)
    v1 = v1 + jnp.uint32((_KS2 + 1) & 0xFFFFFFFF)
    v0, v1 = _threefry_rounds(v0, v1, _ROT1)
    v0 = v0 + jnp.uint32(_KS2)
    v1 = v1 + jnp.uint32((_KS0 + 2) & 0xFFFFFFFF)
    v0, v1 = _threefry_rounds(v0, v1, _ROT0)
    v0 = v0 + jnp.uint32(_KS0)
    v1 = v1 + jnp.uint32((_KS1 + 3) & 0xFFFFFFFF)
    v0, v1 = _threefry_rounds(v0, v1, _ROT1)
    v0 = v0 + jnp.uint32(_KS1)
    v1 = v1 + jnp.uint32((_KS2 + 4) & 0xFFFFFFFF)
    v0, v1 = _threefry_rounds(v0, v1, _ROT0)
    v0 = v0 + jnp.uint32(_KS2)
    v1 = v1 + jnp.uint32((_KS0 + 5) & 0xFFFFFFFF)
    return v0 ^ v1


def _gumbel_tile(cols):
    rows = jax.lax.broadcasted_iota(jnp.int32, cols.shape, 0)
    flat = rows * _N + cols
    bits = _threefry_bits(flat.astype(jnp.uint32))
    fb = jax.lax.shift_right_logical(bits, jnp.uint32(9)) | jnp.uint32(0x3F800000)
    floats = jax.lax.bitcast_convert_type(fb, jnp.float32) - jnp.float32(1.0)
    tiny = jnp.float32(_TINY)
    u = jnp.maximum(tiny, floats * (jnp.float32(1.0) - tiny) + tiny)
    return -jnp.log(-jnp.log(u))


def _body(z_ref, w_ref, b_ref, logp_ref, prop_ref,
          logits_vmem, m_ref, s_ref, ymax_ref, choice_ref):
    i = pl.program_id(0)

    @pl.when(i == 0)
    def _init():
        m_ref[...] = jnp.full((_B, 128), -jnp.inf, jnp.float32)
        s_ref[...] = jnp.zeros((_B, 128), jnp.float32)
        ymax_ref[...] = jnp.full((_B, 128), -jnp.inf, jnp.float32)
        choice_ref[...] = jnp.zeros((_B, 128), jnp.int32)

    @pl.when(i < _NT1)
    def _phase_a():
        logits = jax.lax.dot_general(
            z_ref[...], w_ref[...], (((1,), (1,)), ((), ())),
            preferred_element_type=jnp.float32,
            precision=jax.lax.Precision.DEFAULT)
        logits = logits + b_ref[...][None, :]
        cols = jax.lax.broadcasted_iota(jnp.int32, (_B, _T1), 1) + i * _T1
        logits = jnp.where(cols < _N, logits, -jnp.inf)
        logits_vmem[:, pl.ds(i * _T1, _T1)] = logits

        m_old = m_ref[...]
        tile_max = jnp.max(logits, axis=1, keepdims=True)
        m_new = jnp.maximum(m_old, tile_max)
        s_ref[...] = (s_ref[...] * jnp.exp(m_old - m_new)
                      + jnp.sum(jnp.exp(logits - m_new[:, 0:1]),
                                axis=1, keepdims=True))
        m_ref[...] = m_new

        y = logits + _gumbel_tile(cols)
        ty_max = jnp.max(y, axis=1, keepdims=True)
        t_arg = jnp.min(jnp.where(y == ty_max, cols, jnp.int32(2147483647)),
                        axis=1, keepdims=True)
        upd = ty_max > ymax_ref[...][:, 0:1]
        ymax_ref[...] = jnp.where(upd, ty_max, ymax_ref[...])
        choice_ref[...] = jnp.where(upd, t_arg, choice_ref[...])

    @pl.when(i >= _NT1)
    def _phase_b():
        j = i - _NT1
        lse = m_ref[...][:, 0:1] + jnp.log(s_ref[...][:, 0:1])
        logp_ref[...] = logits_vmem[:, pl.ds(j * _T2, _T2)] - lse
        cols = jax.lax.broadcasted_iota(jnp.int32, (_B, _T2), 1) + j * _T2
        ch = choice_ref[...][:, 0:1]
        prop_ref[...] = jnp.where(
            cols == 0, jnp.float32(0.5),
            jnp.where(cols == ch + 1, jnp.float32(1.0), jnp.float32(0.0)))


def kernel(z, W, b):
    log_p, proposal = pl.pallas_call(
        _body,
        grid=(_NT1 + _NT2,),
        in_specs=[
            pl.BlockSpec((_B, _F), lambda i: (0, 0)),
            pl.BlockSpec((_T1, _F), lambda i: (jnp.minimum(i, _NT1 - 1), 0)),
            pl.BlockSpec((_T1,), lambda i: (jnp.minimum(i, _NT1 - 1),)),
        ],
        out_specs=[
            pl.BlockSpec((_B, _T2), lambda i: (0, jnp.maximum(i - _NT1, 0))),
            pl.BlockSpec((_B, _T2), lambda i: (0, jnp.maximum(i - _NT1, 0))),
        ],
        out_shape=[
            jax.ShapeDtypeStruct((_B, _N), jnp.float32),
            jax.ShapeDtypeStruct((_B, _N + 1), jnp.float32),
        ],
        scratch_shapes=[
            pltpu.VMEM((_B, _SW), jnp.float32),
            pltpu.VMEM((_B, 128), jnp.float32),
            pltpu.VMEM((_B, 128), jnp.float32),
            pltpu.VMEM((_B, 128), jnp.float32),
            pltpu.VMEM((_B, 128), jnp.int32),
        ],
    )(z, W, b)
    return proposal, log_p


# speedup vs baseline: 2.1018x; 1.0016x over previous
import jax
import jax.numpy as jnp
from jax.experimental import pallas as pl
from jax.experimental.pallas import tpu as pltpu

_N = 1000000
_B = 8
_F = 64
_T1 = 8192
_NT1 = (_N + _T1 - 1) // _T1
_T2 = 65536
_NT2 = (_N + 1 + _T2 - 1) // _T2
_SW = _T2 * _NT2

_TINY = 1.1754943508222875e-38

_KS0 = 0
_KS1 = 42
_KS2 = (0x1BD11BDA ^ 42) & 0xFFFFFFFF
_ROT0 = (13, 15, 26, 6)
_ROT1 = (17, 29, 16, 24)


def _rotl(x, d):
    return jax.lax.shift_left(x, jnp.uint32(d)) | jax.lax.shift_right_logical(
        x, jnp.uint32(32 - d))


def _threefry_rounds(v0, v1, rots):
    for r in rots:
        v0 = v0 + v1
        v1 = _rotl(v1, r)
        v1 = v0 ^ v1
    return v0, v1


def _threefry_bits(x1):
    v0 = jnp.zeros_like(x1) + jnp.uint32(_KS0)
    v1 = x1 + jnp.uint32(_KS1)
    v0, v1 = _threefry_rounds(v0, v1, _ROT0)
    v0 = v0 + jnp.uint32(_KS1)
    v1 = v1 + jnp.uint32((_KS2 + 1) & 0xFFFFFFFF)
    v0, v1 = _threefry_rounds(v0, v1, _ROT1)
    v0 = v0 + jnp.uint32(_KS2)
    v1 = v1 + jnp.uint32((_KS0 + 2) & 0xFFFFFFFF)
    v0, v1 = _threefry_rounds(v0, v1, _ROT0)
    v0 = v0 + jnp.uint32(_KS0)
    v1 = v1 + jnp.uint32((_KS1 + 3) & 0xFFFFFFFF)
    v0, v1 = _threefry_rounds(v0, v1, _ROT1)
    v0 = v0 + jnp.uint32(_KS1)
    v1 = v1 + jnp.uint32((_KS2 + 4) & 0xFFFFFFFF)
    v0, v1 = _threefry_rounds(v0, v1, _ROT0)
    v0 = v0 + jnp.uint32(_KS2)
    v1 = v1 + jnp.uint32((_KS0 + 5) & 0xFFFFFFFF)
    return v0 ^ v1


def _gumbel_for(cols):
    rows = jax.lax.broadcasted_iota(jnp.int32, cols.shape, 0)
    flat = rows * _N + cols
    bits = _threefry_bits(flat.astype(jnp.uint32))
    fb = jax.lax.shift_right_logical(bits, jnp.uint32(9)) | jnp.uint32(0x3F800000)
    floats = jax.lax.bitcast_convert_type(fb, jnp.float32) - jnp.float32(1.0)
    tiny = jnp.float32(_TINY)
    u = jnp.maximum(tiny, floats * (jnp.float32(1.0) - tiny) + tiny)
    return -jnp.log(-jnp.log(u))


def _body(z_ref, w_ref, b_ref, logp_ref, prop_ref,
          vbuf, m_ref, s_ref, ymax_ref, choice_ref):
    i = pl.program_id(0)

    @pl.when(i == 0)
    def _init():
        m_ref[...] = jnp.full((_B, 128), -jnp.inf, jnp.float32)
        s_ref[...] = jnp.zeros((_B, 128), jnp.float32)
        ymax_ref[...] = jnp.full((_B, 128), -jnp.inf, jnp.float32)
        choice_ref[...] = jnp.zeros((_B, 128), jnp.int32)

    @pl.when(i < _NT1)
    def _phase_a():
        logits = jax.lax.dot_general(
            z_ref[...], w_ref[...], (((1,), (1,)), ((), ())),
            preferred_element_type=jnp.float32,
            precision=jax.lax.Precision.DEFAULT)
        cols = jax.lax.broadcasted_iota(jnp.int32, (_B, _T1), 1) + i * _T1
        logits = logits + b_ref[...][None, :]
        logits = jnp.where(cols < _N, logits, -jnp.inf)
        vbuf[:, pl.ds(i * _T1, _T1)] = logits

        m_old = m_ref[...]
        tile_max = jnp.max(logits, axis=1, keepdims=True)
        m_new = jnp.maximum(m_old, tile_max)
        s_ref[...] = (s_ref[...] * jnp.exp(m_old - m_new)
                      + jnp.sum(jnp.exp(logits - m_new[:, 0:1]),
                                axis=1, keepdims=True))
        m_ref[...] = m_new

        y = logits + _gumbel_for(cols)
        ty_max = jnp.max(y, axis=1, keepdims=True)
        t_arg = jnp.min(jnp.where(y == ty_max, cols, jnp.int32(2147483647)),
                        axis=1, keepdims=True)
        upd = ty_max > ymax_ref[...][:, 0:1]
        ymax_ref[...] = jnp.where(upd, ty_max, ymax_ref[...])
        choice_ref[...] = jnp.where(upd, t_arg, choice_ref[...])

    @pl.when(i >= _NT1)
    def _phase_b():
        j = i - _NT1
        lse = m_ref[...][:, 0:1] + jnp.log(s_ref[...][:, 0:1])
        logp_ref[...] = vbuf[:, pl.ds(j * _T2, _T2)] - lse
        cols = jax.lax.broadcasted_iota(jnp.int32, (_B, _T2), 1) + j * _T2
        ch = choice_ref[...][:, 0:1]
        prop_ref[...] = jnp.where(
            cols == 0, jnp.float32(0.5),
            jnp.where(cols == ch + 1, jnp.float32(1.0), jnp.float32(0.0)))


def kernel(z, W, b):
    log_p, proposal = pl.pallas_call(
        _body,
        grid=(_NT1 + _NT2,),
        in_specs=[
            pl.BlockSpec((_B, _F), lambda i: (0, 0)),
            pl.BlockSpec((_T1, _F), lambda i: (jnp.minimum(i, _NT1 - 1), 0)),
            pl.BlockSpec((_T1,), lambda i: (jnp.minimum(i, _NT1 - 1),)),
        ],
        out_specs=[
            pl.BlockSpec((_B, _T2), lambda i: (0, jnp.maximum(i - _NT1, 0))),
            pl.BlockSpec((_B, _T2), lambda i: (0, jnp.maximum(i - _NT1, 0))),
        ],
        out_shape=[
            jax.ShapeDtypeStruct((_B, _N), jnp.float32),
            jax.ShapeDtypeStruct((_B, _N + 1), jnp.float32),
        ],
        scratch_shapes=[
            pltpu.VMEM((_B, _SW), jnp.float32),
            pltpu.VMEM((_B, 128), jnp.float32),
            pltpu.VMEM((_B, 128), jnp.float32),
            pltpu.VMEM((_B, 128), jnp.float32),
            pltpu.VMEM((_B, 128), jnp.int32),
        ],
    )(z, W, b)
    return proposal, log_p


# speedup vs baseline: 2.2145x; 1.0536x over previous
import jax
import jax.numpy as jnp
from jax.experimental import pallas as pl
from jax.experimental.pallas import tpu as pltpu

_N = 1000000
_B = 8
_F = 64
_T1 = 16384
_NT1 = (_N + _T1 - 1) // _T1
_T2 = 32768
_NT2 = (_N + 1 + _T2 - 1) // _T2
_SW = _T2 * _NT2

_TINY = 1.1754943508222875e-38

_KS0 = 0
_KS1 = 42
_KS2 = (0x1BD11BDA ^ 42) & 0xFFFFFFFF
_ROT0 = (13, 15, 26, 6)
_ROT1 = (17, 29, 16, 24)


def _rotl(x, d):
    return jax.lax.shift_left(x, jnp.uint32(d)) | jax.lax.shift_right_logical(
        x, jnp.uint32(32 - d))


def _threefry_rounds(v0, v1, rots):
    for r in rots:
        v0 = v0 + v1
        v1 = _rotl(v1, r)
        v1 = v0 ^ v1
    return v0, v1


def _threefry_bits(x1):
    v0 = jnp.zeros_like(x1) + jnp.uint32(_KS0)
    v1 = x1 + jnp.uint32(_KS1)
    v0, v1 = _threefry_rounds(v0, v1, _ROT0)
    v0 = v0 + jnp.uint32(_KS1)
    v1 = v1 + jnp.uint32((_KS2 + 1) & 0xFFFFFFFF)
    v0, v1 = _threefry_rounds(v0, v1, _ROT1)
    v0 = v0 + jnp.uint32(_KS2)
    v1 = v1 + jnp.uint32((_KS0 + 2) & 0xFFFFFFFF)
    v0, v1 = _threefry_rounds(v0, v1, _ROT0)
    v0 = v0 + jnp.uint32(_KS0)
    v1 = v1 + jnp.uint32((_KS1 + 3) & 0xFFFFFFFF)
    v0, v1 = _threefry_rounds(v0, v1, _ROT1)
    v0 = v0 + jnp.uint32(_KS1)
    v1 = v1 + jnp.uint32((_KS2 + 4) & 0xFFFFFFFF)
    v0, v1 = _threefry_rounds(v0, v1, _ROT0)
    v0 = v0 + jnp.uint32(_KS2)
    v1 = v1 + jnp.uint32((_KS0 + 5) & 0xFFFFFFFF)
    return v0 ^ v1


def _gumbel_for(cols):
    rows = jax.lax.broadcasted_iota(jnp.int32, cols.shape, 0)
    flat = rows * _N + cols
    bits = _threefry_bits(flat.astype(jnp.uint32))
    fb = jax.lax.shift_right_logical(bits, jnp.uint32(9)) | jnp.uint32(0x3F800000)
    floats = jax.lax.bitcast_convert_type(fb, jnp.float32) - jnp.float32(1.0)
    tiny = jnp.float32(_TINY)
    u = jnp.maximum(tiny, floats * (jnp.float32(1.0) - tiny) + tiny)
    return -jnp.log(-jnp.log(u))


def _body(z_ref, w_ref, b_ref, logp_ref, prop_ref,
          vbuf, m_ref, s_ref, ymax_ref, choice_ref):
    i = pl.program_id(0)

    @pl.when(i == 0)
    def _init():
        m_ref[...] = jnp.full((_B, 128), -jnp.inf, jnp.float32)
        s_ref[...] = jnp.zeros((_B, 128), jnp.float32)
        ymax_ref[...] = jnp.full((_B, 128), -jnp.inf, jnp.float32)
        choice_ref[...] = jnp.zeros((_B, 128), jnp.int32)

    @pl.when(i < _NT1)
    def _phase_a():
        logits = jax.lax.dot_general(
            z_ref[...], w_ref[...], (((1,), (1,)), ((), ())),
            preferred_element_type=jnp.float32,
            precision=jax.lax.Precision.DEFAULT)
        cols = jax.lax.broadcasted_iota(jnp.int32, (_B, _T1), 1) + i * _T1
        logits = logits + b_ref[...][None, :]
        logits = jnp.where(cols < _N, logits, -jnp.inf)
        vbuf[:, pl.ds(i * _T1, _T1)] = logits

        m_old = m_ref[...]
        tile_max = jnp.max(logits, axis=1, keepdims=True)
        m_new = jnp.maximum(m_old, tile_max)
        s_ref[...] = (s_ref[...] * jnp.exp(m_old - m_new)
                      + jnp.sum(jnp.exp(logits - m_new[:, 0:1]),
                                axis=1, keepdims=True))
        m_ref[...] = m_new

        y = logits + _gumbel_for(cols)
        ty_max = jnp.max(y, axis=1, keepdims=True)
        t_arg = jnp.min(jnp.where(y == ty_max, cols, jnp.int32(2147483647)),
                        axis=1, keepdims=True)
        upd = ty_max > ymax_ref[...][:, 0:1]
        ymax_ref[...] = jnp.where(upd, ty_max, ymax_ref[...])
        choice_ref[...] = jnp.where(upd, t_arg, choice_ref[...])

    @pl.when(i >= _NT1)
    def _phase_b():
        j = i - _NT1
        lse = m_ref[...][:, 0:1] + jnp.log(s_ref[...][:, 0:1])
        logp_ref[...] = vbuf[:, pl.ds(j * _T2, _T2)] - lse
        cols = jax.lax.broadcasted_iota(jnp.int32, (_B, _T2), 1) + j * _T2
        ch = choice_ref[...][:, 0:1]
        prop_ref[...] = jnp.where(
            cols == 0, jnp.float32(0.5),
            jnp.where(cols == ch + 1, jnp.float32(1.0), jnp.float32(0.0)))


def kernel(z, W, b):
    log_p, proposal = pl.pallas_call(
        _body,
        grid=(_NT1 + _NT2,),
        in_specs=[
            pl.BlockSpec((_B, _F), lambda i: (0, 0)),
            pl.BlockSpec((_T1, _F), lambda i: (jnp.minimum(i, _NT1 - 1), 0)),
            pl.BlockSpec((_T1,), lambda i: (jnp.minimum(i, _NT1 - 1),)),
        ],
        out_specs=[
            pl.BlockSpec((_B, _T2), lambda i: (0, jnp.maximum(i - _NT1, 0))),
            pl.BlockSpec((_B, _T2), lambda i: (0, jnp.maximum(i - _NT1, 0))),
        ],
        out_shape=[
            jax.ShapeDtypeStruct((_B, _N), jnp.float32),
            jax.ShapeDtypeStruct((_B, _N + 1), jnp.float32),
        ],
        scratch_shapes=[
            pltpu.VMEM((_B, _SW), jnp.float32),
            pltpu.VMEM((_B, 128), jnp.float32),
            pltpu.VMEM((_B, 128), jnp.float32),
            pltpu.VMEM((_B, 128), jnp.float32),
            pltpu.VMEM((_B, 128), jnp.int32),
        ],
    )(z, W, b)
    return proposal, log_p
